# baseline (speedup 1.0000x reference)
import functools

import jax
import jax.numpy as jnp
from jax import lax
from jax.experimental import pallas as pl
from jax.experimental.pallas import tpu as pltpu
from jax.experimental.pallas import tpu_sc as plsc

_BATCH = 64
_THRESH2 = 3.0
_ALPHA = 2.0
_BETA = 1.0
_PER_BATCH = 256 * 512 * 3
_NCH = 8
_CH = _PER_BATCH // _NCH
_TRI_PER_CH = _CH // 48


def _sc_score(z_hbm, bc_hbm, out_hbm, buf_a, buf_b, bc_v, o_v, sem_a,
              sem_b):
    wid = lax.axis_index("s") * 2 + lax.axis_index("c")
    iota = lax.iota(jnp.int32, 16)
    idx0 = iota * 3
    bufs = [buf_a, buf_b]
    sems = [sem_a, sem_b]

    for bi in range(2):
        b = wid * 2 + bi
        pltpu.sync_copy(bc_hbm.at[b], bc_v)
        bcoef = bc_v[...]

        handles = {0: pltpu.async_copy(z_hbm.at[b, 0], bufs[0], sems[0])}
        acc = jnp.zeros((16,), jnp.float32)
        for c in range(_NCH):
            if c + 1 < _NCH:
                handles[c + 1] = pltpu.async_copy(
                    z_hbm.at[b, c + 1], bufs[(c + 1) % 2], sems[(c + 1) % 2])
            handles[c].wait()
            buf = bufs[c % 2]

            def step(k, acc):
                base = idx0 + k * 48
                x0 = plsc.load_gather(buf, [base])
                x1 = plsc.load_gather(buf, [base + 1])
                x2 = plsc.load_gather(buf, [base + 2])
                z2 = x0 * x0 + x1 * x1 + x2 * x2
                e = jnp.exp(z2 * bcoef)
                return acc + jnp.where(z2 < _THRESH2, e, 0.0)

            acc = lax.fori_loop(0, _TRI_PER_CH, step, acc)
        o_v[...] = acc
        pltpu.sync_copy(o_v, out_hbm.at[b])


def _finish_body(part_ref, r_ref, nobs_ref, raw_ref, mu_ref, s2_ref,
                 obj_ref):
    raw = jnp.sum(part_ref[...], axis=1, keepdims=True)
    r = r_ref[...]
    a = 1.0 / (r * r)
    b = 0.5 * a
    t2 = _THRESH2
    mu = (1.0 - jnp.exp(-b * t2)) / (4.0 * b)
    mean_s2 = (1.0 - jnp.exp(-2.0 * b * t2)) / (8.0 * b)
    sigma2 = mean_s2 - mu * mu
    n = nobs_ref[0, 0]
    mu = n * mu
    sigma2 = n * sigma2
    raw_ref[...] = raw
    mu_ref[...] = mu
    s2_ref[...] = sigma2
    obj_ref[...] = raw - _ALPHA * mu - _BETA * sigma2


@functools.partial(jax.jit, static_argnames=())
def kernel(z, R, num_obs):
    zf = z.reshape(_BATCH, _NCH, _CH)
    bcast = jnp.broadcast_to((-0.5 / (R * R))[:, None], (_BATCH, 16))
    mesh = plsc.VectorSubcoreMesh(core_axis_name="c", subcore_axis_name="s")

    sc_call = pl.kernel(
        _sc_score,
        mesh=mesh,
        compiler_params=pltpu.CompilerParams(needs_layout_passes=False),
        out_type=jax.ShapeDtypeStruct((_BATCH, 16), jnp.float32),
        scratch_types=[
            pltpu.VMEM((_CH,), jnp.float32),
            pltpu.VMEM((_CH,), jnp.float32),
            pltpu.VMEM((16,), jnp.float32),
            pltpu.VMEM((16,), jnp.float32),
            pltpu.SemaphoreType.DMA,
            pltpu.SemaphoreType.DMA,
        ],
    )
    partials = sc_call(zf, bcast)

    nobs = jnp.asarray(num_obs, jnp.float32).reshape(1, 1)
    raw, mu, sigma2, obj = pl.pallas_call(
        _finish_body,
        in_specs=[
            pl.BlockSpec((_BATCH, 16), lambda: (0, 0)),
            pl.BlockSpec((_BATCH, 1), lambda: (0, 0)),
            pl.BlockSpec((1, 1), lambda: (0, 0), memory_space=pltpu.SMEM),
        ],
        out_specs=[pl.BlockSpec((_BATCH, 1), lambda: (0, 0))] * 4,
        out_shape=[jax.ShapeDtypeStruct((_BATCH, 1), jnp.float32)] * 4,
    )(partials, R.reshape(_BATCH, 1), nobs)

    return (raw.reshape(_BATCH), mu.reshape(_BATCH),
            sigma2.reshape(_BATCH), obj.reshape(_BATCH))


# speedup vs baseline: 1.0161x; 1.0161x over previous
import functools

import jax
import jax.numpy as jnp
from jax import lax
from jax.experimental import pallas as pl
from jax.experimental.pallas import tpu as pltpu
from jax.experimental.pallas import tpu_sc as plsc

_BATCH = 64
_THRESH2 = 3.0
_ALPHA = 2.0
_BETA = 1.0
_PER_BATCH = 256 * 512 * 3
_NCH = 8
_CH = _PER_BATCH // _NCH
_TRI_PER_CH = _CH // 48
_UNROLL = 8


def _sc_score(z_hbm, bc_hbm, out_hbm, buf_a, buf_b, bc_v, o_v, sem_a,
              sem_b):
    wid = lax.axis_index("s") * 2 + lax.axis_index("c")
    iota = lax.iota(jnp.int32, 16)
    idx0 = iota * 3
    bufs = [buf_a, buf_b]
    sems = [sem_a, sem_b]

    for bi in range(2):
        b = wid * 2 + bi
        pltpu.sync_copy(bc_hbm.at[b], bc_v)
        bcoef = bc_v[...]

        handles = {0: pltpu.async_copy(z_hbm.at[b, 0], bufs[0], sems[0])}
        acc = (jnp.zeros((16,), jnp.float32),) * 4
        for c in range(_NCH):
            if c + 1 < _NCH:
                handles[c + 1] = pltpu.async_copy(
                    z_hbm.at[b, c + 1], bufs[(c + 1) % 2], sems[(c + 1) % 2])
            handles[c].wait()
            buf = bufs[c % 2]

            def step(k, carry):
                accs = list(carry)
                base = k * (48 * _UNROLL)
                for u in range(_UNROLL):
                    idx = idx0 + (base + 48 * u)
                    x0 = plsc.load_gather(buf, [idx])
                    x1 = plsc.load_gather(buf, [idx + 1])
                    x2 = plsc.load_gather(buf, [idx + 2])
                    z2 = x0 * x0 + x1 * x1 + x2 * x2
                    e = jnp.exp(z2 * bcoef)
                    accs[u % 4] = accs[u % 4] + jnp.where(
                        z2 < _THRESH2, e, 0.0)
                return tuple(accs)

            acc = lax.fori_loop(0, _TRI_PER_CH // _UNROLL, step, acc)
        o_v[...] = (acc[0] + acc[1]) + (acc[2] + acc[3])
        pltpu.sync_copy(o_v, out_hbm.at[b])


def _finish_body(part_ref, r_ref, nobs_ref, raw_ref, mu_ref, s2_ref,
                 obj_ref):
    raw = jnp.sum(part_ref[...], axis=1, keepdims=True)
    r = r_ref[...]
    a = 1.0 / (r * r)
    b = 0.5 * a
    t2 = _THRESH2
    mu = (1.0 - jnp.exp(-b * t2)) / (4.0 * b)
    mean_s2 = (1.0 - jnp.exp(-2.0 * b * t2)) / (8.0 * b)
    sigma2 = mean_s2 - mu * mu
    n = nobs_ref[0, 0]
    mu = n * mu
    sigma2 = n * sigma2
    raw_ref[...] = raw
    mu_ref[...] = mu
    s2_ref[...] = sigma2
    obj_ref[...] = raw - _ALPHA * mu - _BETA * sigma2


@functools.partial(jax.jit, static_argnames=())
def kernel(z, R, num_obs):
    zf = z.reshape(_BATCH, _NCH, _CH)
    bcast = jnp.broadcast_to((-0.5 / (R * R))[:, None], (_BATCH, 16))
    mesh = plsc.VectorSubcoreMesh(core_axis_name="c", subcore_axis_name="s")

    sc_call = pl.kernel(
        _sc_score,
        mesh=mesh,
        compiler_params=pltpu.CompilerParams(needs_layout_passes=False),
        out_type=jax.ShapeDtypeStruct((_BATCH, 16), jnp.float32),
        scratch_types=[
            pltpu.VMEM((_CH,), jnp.float32),
            pltpu.VMEM((_CH,), jnp.float32),
            pltpu.VMEM((16,), jnp.float32),
            pltpu.VMEM((16,), jnp.float32),
            pltpu.SemaphoreType.DMA,
            pltpu.SemaphoreType.DMA,
        ],
    )
    partials = sc_call(zf, bcast)

    nobs = jnp.asarray(num_obs, jnp.float32).reshape(1, 1)
    raw, mu, sigma2, obj = pl.pallas_call(
        _finish_body,
        in_specs=[
            pl.BlockSpec((_BATCH, 16), lambda: (0, 0)),
            pl.BlockSpec((_BATCH, 1), lambda: (0, 0)),
            pl.BlockSpec((1, 1), lambda: (0, 0), memory_space=pltpu.SMEM),
        ],
        out_specs=[pl.BlockSpec((_BATCH, 1), lambda: (0, 0))] * 4,
        out_shape=[jax.ShapeDtypeStruct((_BATCH, 1), jnp.float32)] * 4,
    )(partials, R.reshape(_BATCH, 1), nobs)

    return (raw.reshape(_BATCH), mu.reshape(_BATCH),
            sigma2.reshape(_BATCH), obj.reshape(_BATCH))


# speedup vs baseline: 6.0671x; 5.9707x over previous
import functools

import jax
import jax.numpy as jnp
from jax.experimental import pallas as pl
from jax.experimental.pallas import tpu as pltpu

_BATCH = 64
_THRESH2 = 3.0
_ALPHA = 2.0
_BETA = 1.0
_OBS_R = 256
_OBS_S = 512


def _score_body(z_ref, r_ref, out_ref):
    j = pl.program_id(1)
    x = z_ref[0]
    x0 = x[0]
    x1 = x[1]
    x2 = x[2]
    z2 = x0 * x0 + x1 * x1 + x2 * x2
    b_coef = -0.5 / (r_ref[0, 0, 0] * r_ref[0, 0, 0])
    e = jnp.exp(z2 * b_coef)
    scores = jnp.where(z2 < _THRESH2, e, 0.0)
    ssum = jnp.sum(scores)

    @pl.when(j == 0)
    def _init():
        out_ref[0, 0, 0] = ssum

    @pl.when(j != 0)
    def _acc():
        out_ref[0, 0, 0] += ssum


def _finish_body(raw_ref, r_ref, nobs_ref, mu_ref, s2_ref, obj_ref):
    r = r_ref[...]
    a = 1.0 / (r * r)
    b = 0.5 * a
    t2 = _THRESH2
    mu = (1.0 - jnp.exp(-b * t2)) / (4.0 * b)
    mean_s2 = (1.0 - jnp.exp(-2.0 * b * t2)) / (8.0 * b)
    sigma2 = mean_s2 - mu * mu
    n = nobs_ref[0, 0]
    mu = n * mu
    sigma2 = n * sigma2
    mu_ref[...] = mu
    s2_ref[...] = sigma2
    obj_ref[...] = raw_ref[...] - _ALPHA * mu - _BETA * sigma2


@functools.partial(jax.jit, static_argnames=())
def kernel(z, R, num_obs):
    zt = jnp.transpose(z, (0, 3, 1, 2))

    rb = 128
    nj = _OBS_R // rb
    raw2 = pl.pallas_call(
        _score_body,
        grid=(_BATCH, nj),
        in_specs=[
            pl.BlockSpec((1, 3, rb, _OBS_S), lambda b, j: (b, 0, j, 0)),
            pl.BlockSpec((1, 1, 1), lambda b, j: (b, 0, 0),
                         memory_space=pltpu.SMEM),
        ],
        out_specs=pl.BlockSpec((1, 1, 1), lambda b, j: (b, 0, 0),
                               memory_space=pltpu.SMEM),
        out_shape=jax.ShapeDtypeStruct((_BATCH, 1, 1), jnp.float32),
    )(zt, R.reshape(_BATCH, 1, 1))
    raw = raw2.reshape(_BATCH)

    r2 = R.reshape(1, _BATCH)
    nobs = jnp.asarray(num_obs, jnp.float32).reshape(1, 1)
    mu, sigma2, obj = pl.pallas_call(
        _finish_body,
        in_specs=[
            pl.BlockSpec((1, _BATCH), lambda: (0, 0)),
            pl.BlockSpec((1, _BATCH), lambda: (0, 0)),
            pl.BlockSpec((1, 1), lambda: (0, 0), memory_space=pltpu.SMEM),
        ],
        out_specs=[pl.BlockSpec((1, _BATCH), lambda: (0, 0))] * 3,
        out_shape=[jax.ShapeDtypeStruct((1, _BATCH), jnp.float32)] * 3,
    )(raw.reshape(1, _BATCH), r2, nobs)

    return (raw, mu.reshape(_BATCH), sigma2.reshape(_BATCH),
            obj.reshape(_BATCH))
